# Initial kernel scaffold; baseline (speedup 1.0000x reference)
import jax
import jax.numpy as jnp
from jax.experimental import pallas as pl


def kernel(x, edge_index, edge_weight, h, W1, b1, W2, b2):
    raise NotImplementedError("write your pallas kernel here")


# speedup vs baseline: 13.8777x; 13.8777x over previous
import functools

import jax
import jax.numpy as jnp
from jax import lax
from jax.experimental import pallas as pl
from jax.experimental.pallas import tpu as pltpu
from jax.experimental.pallas import tpu_sc as plsc

_N = 10000
_NP = 10240
_E = 320000
_C = 128
_NC = 2
_NS = 16
_NW = _NC * _NS
_EW = _E // _NW
_CH = 80
_NCHUNK = _EW // _CH
_RT = _NP // _NS


def _mesh():
    return plsc.VectorSubcoreMesh(core_axis_name="c", subcore_axis_name="s")


@functools.cache
def _build_sc_hist():
    return functools.partial(
        pl.kernel,
        out_type=jax.ShapeDtypeStruct((_NC, _NP, _C), jnp.float32),
        mesh=_mesh(),
        scratch_types=[
            pltpu.VMEM((2, _CH), jnp.int32),
            pltpu.VMEM((2, _CH), jnp.int32),
            pltpu.VMEM((_CH, _C), jnp.float32),
            pltpu.VMEM_SHARED((_NP, _C), jnp.float32),
            pltpu.SemaphoreType.DMA,
            pltpu.SemaphoreType.DMA,
        ],
    )(_sc_hist_body)


def _sc_hist_body(ei_h, z128_h, out_h, eb_a, eb_b, ones_v, acc, sem_a, sem_b):
    c = lax.axis_index("c")
    s = lax.axis_index("s")
    w = c * _NS + s
    base = w * _NCHUNK

    def fill(t, _):
        i = t // (_C // 16)
        j = t % (_C // 16)
        ones_v[i, pl.ds(j * 16, 16)] = jnp.full((16,), 1.0, jnp.float32)
        return 0
    lax.fori_loop(0, _CH * (_C // 16), fill, 0)

    pltpu.sync_copy(z128_h, acc.at[pl.ds(s * _RT, _RT)])
    plsc.subcore_barrier()

    ebs = [eb_a, eb_b]
    sems = [sem_a, sem_b]
    cps = [None, None]

    def start_idx(j, b):
        cps[b] = pltpu.async_copy(ei_h.at[base + j], ebs[b], sems[b])

    start_idx(0, 0)
    for j in range(_NCHUNK):
        cur = j % 2
        if j + 1 < _NCHUNK:
            start_idx(j + 1, 1 - cur)
        cps[cur].wait()
        pltpu.sync_copy(ones_v, acc.at[ebs[cur].at[1]], add=True)

    plsc.subcore_barrier()
    pltpu.sync_copy(acc.at[pl.ds(s * _RT, _RT)],
                    out_h.at[c, pl.ds(s * _RT, _RT)])


@functools.cache
def _build_sc_prop():
    return functools.partial(
        pl.kernel,
        out_type=jax.ShapeDtypeStruct((_NC, _NP, _C), jnp.float32),
        mesh=_mesh(),
        scratch_types=[
            pltpu.VMEM((2, _CH), jnp.int32),
            pltpu.VMEM((2, _CH), jnp.int32),
            pltpu.VMEM((_CH, _C), jnp.float32),
            pltpu.VMEM((_CH, _C), jnp.float32),
            pltpu.VMEM_SHARED((_NP, _C), jnp.float32),
            pltpu.SemaphoreType.DMA,
            pltpu.SemaphoreType.DMA,
            pltpu.SemaphoreType.DMA,
            pltpu.SemaphoreType.DMA,
        ],
    )(_sc_prop_body)


def _sc_prop_body(table_h, ei_h, z128_h, out_h,
                  eb_a, eb_b, rw_a, rw_b, acc,
                  sem_ia, sem_ib, sem_ga, sem_gb):
    c = lax.axis_index("c")
    s = lax.axis_index("s")
    w = c * _NS + s
    base = w * _NCHUNK

    pltpu.sync_copy(z128_h, acc.at[pl.ds(s * _RT, _RT)])
    plsc.subcore_barrier()

    ebs = [eb_a, eb_b]
    rws = [rw_a, rw_b]
    isems = [sem_ia, sem_ib]
    gsems = [sem_ga, sem_gb]
    icps = [None, None]
    gcps = [None, None]

    def start_idx(j, b):
        icps[b] = pltpu.async_copy(ei_h.at[base + j], ebs[b], isems[b])

    def start_gather(b):
        gcps[b] = pltpu.async_copy(table_h.at[ebs[b].at[0]], rws[b], gsems[b])

    start_idx(0, 0)
    icps[0].wait()
    start_gather(0)
    if _NCHUNK > 1:
        start_idx(1, 1)
    for j in range(_NCHUNK):
        cur = j % 2
        nxt = 1 - cur
        if j + 1 < _NCHUNK:
            icps[nxt].wait()
            start_gather(nxt)
        gcps[cur].wait()
        pltpu.sync_copy(rws[cur], acc.at[ebs[cur].at[1]], add=True)
        if j + 2 < _NCHUNK:
            start_idx(j + 2, cur)

    plsc.subcore_barrier()
    pltpu.sync_copy(acc.at[pl.ds(s * _RT, _RT)],
                    out_h.at[c, pl.ds(s * _RT, _RT)])


_BLK = 1000
_GRID = _N // _BLK


def _deg_dinv(hist_ref):
    deg = hist_ref[0][:, 0:1] + hist_ref[1][:, 0:1] + 2.0
    return deg, lax.rsqrt(deg)


def _tc_scale_body(hist_ref, x_ref, h_ref, xs_ref, hs_ref):
    _, dinv = _deg_dinv(hist_ref)
    xs_ref[...] = x_ref[...] * dinv
    hs_ref[...] = h_ref[...] * dinv


def _tc_scale(hist2, x, h):
    return pl.pallas_call(
        _tc_scale_body,
        grid=(_GRID,),
        in_specs=[
            pl.BlockSpec((_NC, _BLK, _C), lambda i: (0, i, 0)),
            pl.BlockSpec((_BLK, _C), lambda i: (i, 0)),
            pl.BlockSpec((_BLK, _C), lambda i: (i, 0)),
        ],
        out_specs=[
            pl.BlockSpec((_BLK, _C), lambda i: (i, 0)),
            pl.BlockSpec((_BLK, _C), lambda i: (i, 0)),
        ],
        out_shape=[
            jax.ShapeDtypeStruct((_N, _C), jnp.float32),
            jax.ShapeDtypeStruct((_N, _C), jnp.float32),
        ],
    )(hist2, x, h)


def _tc_conv1_body(px_ref, ph_ref, hist_ref, x_ref, h_ref, w1_ref, b1_ref,
                   ru_ref, mx_ref):
    deg, dinv = _deg_dinv(hist_ref)
    two_over = 2.0 / deg
    mx = dinv * (px_ref[0] + px_ref[1]) + two_over * x_ref[...]
    mh = dinv * (ph_ref[0] + ph_ref[1]) + two_over * h_ref[...]
    cat = jnp.concatenate([mx, mh], axis=1)
    agg = jnp.dot(cat, w1_ref[...], preferred_element_type=jnp.float32)
    ru_ref[...] = jax.nn.sigmoid(agg + b1_ref[...])
    mx_ref[...] = mx


def _tc_conv1(pxp, php, hist2, x, h, W1, b1):
    return pl.pallas_call(
        _tc_conv1_body,
        grid=(_GRID,),
        in_specs=[
            pl.BlockSpec((_NC, _BLK, _C), lambda i: (0, i, 0)),
            pl.BlockSpec((_NC, _BLK, _C), lambda i: (0, i, 0)),
            pl.BlockSpec((_NC, _BLK, _C), lambda i: (0, i, 0)),
            pl.BlockSpec((_BLK, _C), lambda i: (i, 0)),
            pl.BlockSpec((_BLK, _C), lambda i: (i, 0)),
            pl.BlockSpec((2 * _C, 2 * _C), lambda i: (0, 0)),
            pl.BlockSpec((1, 2 * _C), lambda i: (0, 0)),
        ],
        out_specs=[
            pl.BlockSpec((_BLK, 2 * _C), lambda i: (i, 0)),
            pl.BlockSpec((_BLK, _C), lambda i: (i, 0)),
        ],
        out_shape=[
            jax.ShapeDtypeStruct((_N, 2 * _C), jnp.float32),
            jax.ShapeDtypeStruct((_N, _C), jnp.float32),
        ],
    )(pxp, php, hist2, x, h, W1, b1)


def _tc_rhs_body(hist_ref, r_ref, h_ref, rhs_ref):
    _, dinv = _deg_dinv(hist_ref)
    rhs_ref[...] = dinv * (r_ref[...] * h_ref[...])


def _tc_rhs(hist2, r, h):
    return pl.pallas_call(
        _tc_rhs_body,
        grid=(_GRID,),
        in_specs=[
            pl.BlockSpec((_NC, _BLK, _C), lambda i: (0, i, 0)),
            pl.BlockSpec((_BLK, _C), lambda i: (i, 0)),
            pl.BlockSpec((_BLK, _C), lambda i: (i, 0)),
        ],
        out_specs=pl.BlockSpec((_BLK, _C), lambda i: (i, 0)),
        out_shape=jax.ShapeDtypeStruct((_N, _C), jnp.float32),
    )(hist2, r, h)


def _tc_conv2_body(prh_ref, hist_ref, r_ref, h_ref, u_ref, mx_ref, w2_ref,
                   b2_ref, out_ref):
    deg, dinv = _deg_dinv(hist_ref)
    two_over = 2.0 / deg
    rh = r_ref[...] * h_ref[...]
    mrh = dinv * (prh_ref[0] + prh_ref[1]) + two_over * rh
    cat = jnp.concatenate([mx_ref[...], mrh], axis=1)
    agg = jnp.dot(cat, w2_ref[...], preferred_element_type=jnp.float32)
    cnew = jnp.tanh(agg + b2_ref[...])
    u = u_ref[...]
    out_ref[...] = u * h_ref[...] + (1.0 - u) * cnew


def _tc_conv2(prhp, hist2, r, h, u, mx, W2, b2):
    return pl.pallas_call(
        _tc_conv2_body,
        grid=(_GRID,),
        in_specs=[
            pl.BlockSpec((_NC, _BLK, _C), lambda i: (0, i, 0)),
            pl.BlockSpec((_NC, _BLK, _C), lambda i: (0, i, 0)),
            pl.BlockSpec((_BLK, _C), lambda i: (i, 0)),
            pl.BlockSpec((_BLK, _C), lambda i: (i, 0)),
            pl.BlockSpec((_BLK, _C), lambda i: (i, 0)),
            pl.BlockSpec((_BLK, _C), lambda i: (i, 0)),
            pl.BlockSpec((2 * _C, _C), lambda i: (0, 0)),
            pl.BlockSpec((1, _C), lambda i: (0, 0)),
        ],
        out_specs=pl.BlockSpec((_BLK, _C), lambda i: (i, 0)),
        out_shape=jax.ShapeDtypeStruct((_N, _C), jnp.float32),
    )(prhp, hist2, r, h, u, mx, W2, b2)


def kernel(x, edge_index, edge_weight, h, W1, b1, W2, b2):
    del edge_weight
    src = edge_index[0]
    dst = edge_index[1]
    ei3 = jnp.stack([src.reshape(_E // _CH, _CH),
                     dst.reshape(_E // _CH, _CH)], axis=1)
    z128 = jnp.zeros((_RT, _C), jnp.float32)
    b1r = b1.reshape(1, 2 * _C)
    b2r = b2.reshape(1, _C)

    sc_hist = _build_sc_hist()
    sc_prop = _build_sc_prop()
    hist2 = sc_hist(ei3, z128)
    xs, hs = _tc_scale(hist2, x, h)
    pxp = sc_prop(xs, ei3, z128)
    php = sc_prop(hs, ei3, z128)
    ru, mx = _tc_conv1(pxp, php, hist2, x, h, W1, b1r)
    r = ru[:_N // 2].reshape(_N, _C)
    u = ru[_N // 2:].reshape(_N, _C)
    rhs = _tc_rhs(hist2, r, h)
    prhp = sc_prop(rhs, ei3, z128)
    return _tc_conv2(prhp, hist2, r, h, u, mx, W2, b2r)


# speedup vs baseline: 14.4810x; 1.0435x over previous
import functools

import jax
import jax.numpy as jnp
from jax import lax
from jax.experimental import pallas as pl
from jax.experimental.pallas import tpu as pltpu
from jax.experimental.pallas import tpu_sc as plsc

_N = 10000
_NP = 10240
_E = 320000
_C = 128
_NC = 2
_NS = 16
_NW = _NC * _NS
_EW = _E // _NW
_CH = 80
_NCHUNK = _EW // _CH
_RT = _NP // _NS
_NB = 4


def _mesh():
    return plsc.VectorSubcoreMesh(core_axis_name="c", subcore_axis_name="s")


@functools.cache
def _build_sc_hist():
    return functools.partial(
        pl.kernel,
        out_type=jax.ShapeDtypeStruct((_NC, _NP, _C), jnp.float32),
        mesh=_mesh(),
        scratch_types=[
            pltpu.VMEM((2, _CH), jnp.int32),
            pltpu.VMEM((2, _CH), jnp.int32),
            pltpu.VMEM((_CH, _C), jnp.float32),
            pltpu.VMEM_SHARED((_NP, _C), jnp.float32),
            pltpu.SemaphoreType.DMA,
            pltpu.SemaphoreType.DMA,
        ],
    )(_sc_hist_body)


def _sc_hist_body(ei_h, z128_h, out_h, eb_a, eb_b, ones_v, acc, sem_a, sem_b):
    c = lax.axis_index("c")
    s = lax.axis_index("s")
    w = c * _NS + s
    base = w * _NCHUNK

    def fill(t, _):
        i = t // (_C // 16)
        j = t % (_C // 16)
        ones_v[i, pl.ds(j * 16, 16)] = jnp.full((16,), 1.0, jnp.float32)
        return 0
    lax.fori_loop(0, _CH * (_C // 16), fill, 0)

    pltpu.sync_copy(z128_h, acc.at[pl.ds(s * _RT, _RT)])
    plsc.subcore_barrier()

    ebs = [eb_a, eb_b]
    sems = [sem_a, sem_b]
    cps = [None, None]

    def start_idx(j, b):
        cps[b] = pltpu.async_copy(ei_h.at[base + j], ebs[b], sems[b])

    start_idx(0, 0)
    for j in range(_NCHUNK):
        cur = j % 2
        if j + 1 < _NCHUNK:
            start_idx(j + 1, 1 - cur)
        cps[cur].wait()
        pltpu.sync_copy(ones_v, acc.at[ebs[cur].at[1]], add=True)

    plsc.subcore_barrier()
    pltpu.sync_copy(acc.at[pl.ds(s * _RT, _RT)],
                    out_h.at[c, pl.ds(s * _RT, _RT)])


@functools.cache
def _build_sc_prop():
    return functools.partial(
        pl.kernel,
        out_type=jax.ShapeDtypeStruct((_NC, _NP, _C), jnp.float32),
        mesh=_mesh(),
        scratch_types=(
            [pltpu.VMEM((2, _CH), jnp.int32) for _ in range(_NB)]
            + [pltpu.VMEM((_CH, _C), jnp.float32) for _ in range(_NB)]
            + [pltpu.VMEM_SHARED((_NP, _C), jnp.float32)]
            + [pltpu.SemaphoreType.DMA for _ in range(2 * _NB)]
        ),
    )(_sc_prop_body)


def _sc_prop_body(table_h, ei_h, z128_h, out_h, *refs):
    ebs = list(refs[:_NB])
    rws = list(refs[_NB:2 * _NB])
    acc = refs[2 * _NB]
    isems = list(refs[2 * _NB + 1:2 * _NB + 1 + _NB])
    gsems = list(refs[2 * _NB + 1 + _NB:])
    c = lax.axis_index("c")
    s = lax.axis_index("s")
    w = c * _NS + s
    base = w * _NCHUNK

    pltpu.sync_copy(z128_h, acc.at[pl.ds(s * _RT, _RT)])
    plsc.subcore_barrier()

    icps = [None] * _NB
    gcps = [None] * _NB

    def start_idx(j, b):
        icps[b] = pltpu.async_copy(ei_h.at[base + j], ebs[b], isems[b])

    def start_gather(b):
        gcps[b] = pltpu.async_copy(table_h.at[ebs[b].at[0]], rws[b], gsems[b])

    for b in range(min(_NB, _NCHUNK)):
        start_idx(b, b)
    for b in range(min(_NB - 1, _NCHUNK)):
        icps[b].wait()
        start_gather(b)
    for j in range(_NCHUNK):
        cur = j % _NB
        if j + _NB - 1 < _NCHUNK:
            nb = (j + _NB - 1) % _NB
            icps[nb].wait()
            start_gather(nb)
        gcps[cur].wait()
        pltpu.sync_copy(rws[cur], acc.at[ebs[cur].at[1]], add=True)
        if j + _NB < _NCHUNK:
            start_idx(j + _NB, cur)

    plsc.subcore_barrier()
    pltpu.sync_copy(acc.at[pl.ds(s * _RT, _RT)],
                    out_h.at[c, pl.ds(s * _RT, _RT)])


_BLK = 1000
_GRID = _N // _BLK


def _deg_dinv(hist_ref):
    deg = hist_ref[0][:, 0:1] + hist_ref[1][:, 0:1] + 2.0
    return deg, lax.rsqrt(deg)


def _tc_scale_body(hist_ref, x_ref, h_ref, xs_ref, hs_ref):
    _, dinv = _deg_dinv(hist_ref)
    xs_ref[...] = x_ref[...] * dinv
    hs_ref[...] = h_ref[...] * dinv


def _tc_scale(hist2, x, h):
    return pl.pallas_call(
        _tc_scale_body,
        grid=(_GRID,),
        in_specs=[
            pl.BlockSpec((_NC, _BLK, _C), lambda i: (0, i, 0)),
            pl.BlockSpec((_BLK, _C), lambda i: (i, 0)),
            pl.BlockSpec((_BLK, _C), lambda i: (i, 0)),
        ],
        out_specs=[
            pl.BlockSpec((_BLK, _C), lambda i: (i, 0)),
            pl.BlockSpec((_BLK, _C), lambda i: (i, 0)),
        ],
        out_shape=[
            jax.ShapeDtypeStruct((_N, _C), jnp.float32),
            jax.ShapeDtypeStruct((_N, _C), jnp.float32),
        ],
    )(hist2, x, h)


def _tc_conv1_body(px_ref, ph_ref, hist_ref, x_ref, h_ref, w1_ref, b1_ref,
                   ru_ref, mx_ref):
    deg, dinv = _deg_dinv(hist_ref)
    two_over = 2.0 / deg
    mx = dinv * (px_ref[0] + px_ref[1]) + two_over * x_ref[...]
    mh = dinv * (ph_ref[0] + ph_ref[1]) + two_over * h_ref[...]
    cat = jnp.concatenate([mx, mh], axis=1)
    agg = jnp.dot(cat, w1_ref[...], preferred_element_type=jnp.float32)
    ru_ref[...] = jax.nn.sigmoid(agg + b1_ref[...])
    mx_ref[...] = mx


def _tc_conv1(pxp, php, hist2, x, h, W1, b1):
    return pl.pallas_call(
        _tc_conv1_body,
        grid=(_GRID,),
        in_specs=[
            pl.BlockSpec((_NC, _BLK, _C), lambda i: (0, i, 0)),
            pl.BlockSpec((_NC, _BLK, _C), lambda i: (0, i, 0)),
            pl.BlockSpec((_NC, _BLK, _C), lambda i: (0, i, 0)),
            pl.BlockSpec((_BLK, _C), lambda i: (i, 0)),
            pl.BlockSpec((_BLK, _C), lambda i: (i, 0)),
            pl.BlockSpec((2 * _C, 2 * _C), lambda i: (0, 0)),
            pl.BlockSpec((1, 2 * _C), lambda i: (0, 0)),
        ],
        out_specs=[
            pl.BlockSpec((_BLK, 2 * _C), lambda i: (i, 0)),
            pl.BlockSpec((_BLK, _C), lambda i: (i, 0)),
        ],
        out_shape=[
            jax.ShapeDtypeStruct((_N, 2 * _C), jnp.float32),
            jax.ShapeDtypeStruct((_N, _C), jnp.float32),
        ],
    )(pxp, php, hist2, x, h, W1, b1)


def _tc_rhs_body(hist_ref, r_ref, h_ref, rhs_ref):
    _, dinv = _deg_dinv(hist_ref)
    rhs_ref[...] = dinv * (r_ref[...] * h_ref[...])


def _tc_rhs(hist2, r, h):
    return pl.pallas_call(
        _tc_rhs_body,
        grid=(_GRID,),
        in_specs=[
            pl.BlockSpec((_NC, _BLK, _C), lambda i: (0, i, 0)),
            pl.BlockSpec((_BLK, _C), lambda i: (i, 0)),
            pl.BlockSpec((_BLK, _C), lambda i: (i, 0)),
        ],
        out_specs=pl.BlockSpec((_BLK, _C), lambda i: (i, 0)),
        out_shape=jax.ShapeDtypeStruct((_N, _C), jnp.float32),
    )(hist2, r, h)


def _tc_conv2_body(prh_ref, hist_ref, r_ref, h_ref, u_ref, mx_ref, w2_ref,
                   b2_ref, out_ref):
    deg, dinv = _deg_dinv(hist_ref)
    two_over = 2.0 / deg
    rh = r_ref[...] * h_ref[...]
    mrh = dinv * (prh_ref[0] + prh_ref[1]) + two_over * rh
    cat = jnp.concatenate([mx_ref[...], mrh], axis=1)
    agg = jnp.dot(cat, w2_ref[...], preferred_element_type=jnp.float32)
    cnew = jnp.tanh(agg + b2_ref[...])
    u = u_ref[...]
    out_ref[...] = u * h_ref[...] + (1.0 - u) * cnew


def _tc_conv2(prhp, hist2, r, h, u, mx, W2, b2):
    return pl.pallas_call(
        _tc_conv2_body,
        grid=(_GRID,),
        in_specs=[
            pl.BlockSpec((_NC, _BLK, _C), lambda i: (0, i, 0)),
            pl.BlockSpec((_NC, _BLK, _C), lambda i: (0, i, 0)),
            pl.BlockSpec((_BLK, _C), lambda i: (i, 0)),
            pl.BlockSpec((_BLK, _C), lambda i: (i, 0)),
            pl.BlockSpec((_BLK, _C), lambda i: (i, 0)),
            pl.BlockSpec((_BLK, _C), lambda i: (i, 0)),
            pl.BlockSpec((2 * _C, _C), lambda i: (0, 0)),
            pl.BlockSpec((1, _C), lambda i: (0, 0)),
        ],
        out_specs=pl.BlockSpec((_BLK, _C), lambda i: (i, 0)),
        out_shape=jax.ShapeDtypeStruct((_N, _C), jnp.float32),
    )(prhp, hist2, r, h, u, mx, W2, b2)


def kernel(x, edge_index, edge_weight, h, W1, b1, W2, b2):
    del edge_weight
    src = edge_index[0]
    dst = edge_index[1]
    ei3 = jnp.stack([src.reshape(_E // _CH, _CH),
                     dst.reshape(_E // _CH, _CH)], axis=1)
    z128 = jnp.zeros((_RT, _C), jnp.float32)
    b1r = b1.reshape(1, 2 * _C)
    b2r = b2.reshape(1, _C)

    sc_hist = _build_sc_hist()
    sc_prop = _build_sc_prop()
    hist2 = sc_hist(ei3, z128)
    xs, hs = _tc_scale(hist2, x, h)
    pxp = sc_prop(xs, ei3, z128)
    php = sc_prop(hs, ei3, z128)
    ru, mx = _tc_conv1(pxp, php, hist2, x, h, W1, b1r)
    r = ru[:_N // 2].reshape(_N, _C)
    u = ru[_N // 2:].reshape(_N, _C)
    rhs = _tc_rhs(hist2, r, h)
    prhp = sc_prop(rhs, ei3, z128)
    return _tc_conv2(prhp, hist2, r, h, u, mx, W2, b2r)


# speedup vs baseline: 15.9885x; 1.1041x over previous
import functools

import jax
import jax.numpy as jnp
from jax import lax
from jax.experimental import pallas as pl
from jax.experimental.pallas import tpu as pltpu
from jax.experimental.pallas import tpu_sc as plsc

_N = 10000
_NP = 10240
_E = 320000
_C = 128
_NC = 2
_NS = 16
_NW = _NC * _NS
_EW = _E // _NW
_CH = 80
_NCHUNK = _EW // _CH
_RT = _NP // _NS
_NB = 4


def _mesh():
    return plsc.VectorSubcoreMesh(core_axis_name="c", subcore_axis_name="s")


@functools.cache
def _build_sc_hist():
    return functools.partial(
        pl.kernel,
        out_type=jax.ShapeDtypeStruct((_NC, _NP, _C), jnp.float32),
        mesh=_mesh(),
        scratch_types=[
            pltpu.VMEM((2, _CH), jnp.int32),
            pltpu.VMEM((2, _CH), jnp.int32),
            pltpu.VMEM((_CH, _C), jnp.float32),
            pltpu.VMEM_SHARED((_NP, _C), jnp.float32),
            pltpu.SemaphoreType.DMA,
            pltpu.SemaphoreType.DMA,
        ],
    )(_sc_hist_body)


def _sc_hist_body(ei_h, z128_h, out_h, eb_a, eb_b, ones_v, acc, sem_a, sem_b):
    c = lax.axis_index("c")
    s = lax.axis_index("s")
    w = c * _NS + s
    base = w * _NCHUNK

    def fill(t, _):
        i = t // (_C // 16)
        j = t % (_C // 16)
        ones_v[i, pl.ds(j * 16, 16)] = jnp.full((16,), 1.0, jnp.float32)
        return 0
    lax.fori_loop(0, _CH * (_C // 16), fill, 0)

    pltpu.sync_copy(z128_h, acc.at[pl.ds(s * _RT, _RT)])
    plsc.subcore_barrier()

    ebs = [eb_a, eb_b]
    sems = [sem_a, sem_b]
    cps = [None, None]

    def start_idx(j, b):
        cps[b] = pltpu.async_copy(ei_h.at[base + j], ebs[b], sems[b])

    start_idx(0, 0)
    for j in range(_NCHUNK):
        cur = j % 2
        if j + 1 < _NCHUNK:
            start_idx(j + 1, 1 - cur)
        cps[cur].wait()
        pltpu.sync_copy(ones_v, acc.at[ebs[cur].at[1]], add=True)

    plsc.subcore_barrier()
    pltpu.sync_copy(acc.at[pl.ds(s * _RT, _RT)],
                    out_h.at[c, pl.ds(s * _RT, _RT)])


@functools.cache
def _build_sc_prop():
    return functools.partial(
        pl.kernel,
        out_type=jax.ShapeDtypeStruct((_NC, _NP, _C), jnp.float32),
        mesh=_mesh(),
        scratch_types=(
            [pltpu.VMEM((2, _CH), jnp.int32) for _ in range(_NB)]
            + [pltpu.VMEM((_CH, _C), jnp.float32) for _ in range(_NB)]
            + [pltpu.VMEM_SHARED((_NP, _C), jnp.float32)]
            + [pltpu.SemaphoreType.DMA for _ in range(2 * _NB)]
        ),
    )(_sc_prop_body)


def _sc_prop_body(table_h, ei_h, z128_h, out_h, *refs):
    ebs = list(refs[:_NB])
    rws = list(refs[_NB:2 * _NB])
    acc = refs[2 * _NB]
    isems = list(refs[2 * _NB + 1:2 * _NB + 1 + _NB])
    gsems = list(refs[2 * _NB + 1 + _NB:])
    c = lax.axis_index("c")
    s = lax.axis_index("s")
    w = c * _NS + s
    base = w * _NCHUNK

    pltpu.sync_copy(z128_h, acc.at[pl.ds(s * _RT, _RT)])
    plsc.subcore_barrier()

    icps = [None] * _NB
    gcps = [None] * _NB

    def start_idx(j, b):
        icps[b] = pltpu.async_copy(ei_h.at[base + j], ebs[b], isems[b])

    def start_gather(b):
        gcps[b] = pltpu.async_copy(table_h.at[ebs[b].at[0]], rws[b], gsems[b])

    for b in range(min(_NB, _NCHUNK)):
        start_idx(b, b)
    for b in range(min(_NB - 1, _NCHUNK)):
        icps[b].wait()
        start_gather(b)
    for j in range(_NCHUNK):
        cur = j % _NB
        if j + _NB - 1 < _NCHUNK:
            nb = (j + _NB - 1) % _NB
            icps[nb].wait()
            start_gather(nb)
        gcps[cur].wait()
        pltpu.sync_copy(rws[cur], acc.at[ebs[cur].at[1]], add=True)
        if j + _NB < _NCHUNK:
            start_idx(j + _NB, cur)

    plsc.subcore_barrier()
    pltpu.sync_copy(acc.at[pl.ds(s * _RT, _RT)],
                    out_h.at[c, pl.ds(s * _RT, _RT)])


_CH2 = 160
_NCH2 = _E // _CH2 // _NS


@functools.cache
def _build_sc_prop2():
    return functools.partial(
        pl.kernel,
        out_type=jax.ShapeDtypeStruct((_NC, _NP, _C), jnp.float32),
        mesh=_mesh(),
        scratch_types=(
            [pltpu.VMEM((_CH2,), jnp.int32) for _ in range(2)]
            + [pltpu.VMEM((_CH2,), jnp.int32) for _ in range(2)]
            + [pltpu.VMEM((_CH2,), jnp.int32) for _ in range(2)]
            + [pltpu.VMEM((_CH2, _C), jnp.float32) for _ in range(2)]
            + [pltpu.VMEM_SHARED((_NP, _C), jnp.float32)]
            + [pltpu.SemaphoreType.DMA for _ in range(6)]
        ),
    )(_sc_prop2_body)


def _sc_prop2_body(table2_h, src_h, dst_h, z128_h, out_h, *refs):
    sbs = list(refs[0:2])
    s2s = list(refs[2:4])
    dbs = list(refs[4:6])
    rws = list(refs[6:8])
    acc = refs[8]
    ssems = list(refs[9:11])
    dsems = list(refs[11:13])
    gsems = list(refs[13:15])
    c = lax.axis_index("c")
    s = lax.axis_index("s")
    base = s * _NCH2
    off = c * _N

    pltpu.sync_copy(z128_h, acc.at[pl.ds(s * _RT, _RT)])
    plsc.subcore_barrier()

    scps = [None, None]
    dcps = [None, None]
    gcps = [None, None]

    def start_idx(j, b):
        eoff = pl.multiple_of((base + j) * _CH2, 8)
        scps[b] = pltpu.async_copy(src_h.at[pl.ds(eoff, _CH2)], sbs[b],
                                   ssems[b])
        dcps[b] = pltpu.async_copy(dst_h.at[pl.ds(eoff, _CH2)], dbs[b],
                                   dsems[b])

    def adjust(b):
        scps[b].wait()
        for k in range(_CH2 // 16):
            s2s[b][pl.ds(k * 16, 16)] = sbs[b][pl.ds(k * 16, 16)] + off

    def start_gather(b):
        gcps[b] = pltpu.async_copy(table2_h.at[s2s[b]], rws[b], gsems[b])

    start_idx(0, 0)
    adjust(0)
    start_gather(0)
    start_idx(1, 1)
    for j in range(_NCH2):
        cur = j % 2
        nxt = 1 - cur
        if j + 1 < _NCH2:
            adjust(nxt)
            start_gather(nxt)
        gcps[cur].wait()
        dcps[cur].wait()
        pltpu.sync_copy(rws[cur], acc.at[dbs[cur]], add=True)
        if j + 2 < _NCH2:
            start_idx(j + 2, cur)

    plsc.subcore_barrier()
    pltpu.sync_copy(acc.at[pl.ds(s * _RT, _RT)],
                    out_h.at[c, pl.ds(s * _RT, _RT)])


_BLK = 1000
_GRID = _N // _BLK


def _deg_dinv(hist_ref):
    cnt = (hist_ref[0][:, 0:1] + hist_ref[1][:, 0:1]).astype(jnp.float32)
    deg = cnt + 2.0
    return deg, lax.rsqrt(deg)


def _tc_scale_body(hist_ref, x_ref, h_ref, xs_ref, hs_ref):
    _, dinv = _deg_dinv(hist_ref)
    xs_ref[...] = x_ref[...] * dinv
    hs_ref[...] = h_ref[...] * dinv


def _tc_scale(hist2, x, h):
    return pl.pallas_call(
        _tc_scale_body,
        grid=(_GRID,),
        in_specs=[
            pl.BlockSpec((_NC, _BLK, _C), lambda i: (0, i, 0)),
            pl.BlockSpec((_BLK, _C), lambda i: (i, 0)),
            pl.BlockSpec((_BLK, _C), lambda i: (i, 0)),
        ],
        out_specs=[
            pl.BlockSpec((_BLK, _C), lambda i: (i, 0)),
            pl.BlockSpec((_BLK, _C), lambda i: (i, 0)),
        ],
        out_shape=[
            jax.ShapeDtypeStruct((_N, _C), jnp.float32),
            jax.ShapeDtypeStruct((_N, _C), jnp.float32),
        ],
    )(hist2, x, h)


def _tc_conv1_body(pxh_ref, hist_ref, x_ref, h_ref, w1_ref, b1_ref,
                   ru_ref, mx_ref):
    deg, dinv = _deg_dinv(hist_ref)
    two_over = 2.0 / deg
    mx = dinv * pxh_ref[0] + two_over * x_ref[...]
    mh = dinv * pxh_ref[1] + two_over * h_ref[...]
    cat = jnp.concatenate([mx, mh], axis=1)
    agg = jnp.dot(cat, w1_ref[...], preferred_element_type=jnp.float32)
    ru_ref[...] = jax.nn.sigmoid(agg + b1_ref[...])
    mx_ref[...] = mx


def _tc_conv1(pxh, hist2, x, h, W1, b1):
    return pl.pallas_call(
        _tc_conv1_body,
        grid=(_GRID,),
        in_specs=[
            pl.BlockSpec((_NC, _BLK, _C), lambda i: (0, i, 0)),
            pl.BlockSpec((_NC, _BLK, _C), lambda i: (0, i, 0)),
            pl.BlockSpec((_BLK, _C), lambda i: (i, 0)),
            pl.BlockSpec((_BLK, _C), lambda i: (i, 0)),
            pl.BlockSpec((2 * _C, 2 * _C), lambda i: (0, 0)),
            pl.BlockSpec((1, 2 * _C), lambda i: (0, 0)),
        ],
        out_specs=[
            pl.BlockSpec((_BLK, 2 * _C), lambda i: (i, 0)),
            pl.BlockSpec((_BLK, _C), lambda i: (i, 0)),
        ],
        out_shape=[
            jax.ShapeDtypeStruct((_N, 2 * _C), jnp.float32),
            jax.ShapeDtypeStruct((_N, _C), jnp.float32),
        ],
    )(pxh, hist2, x, h, W1, b1)


def _tc_rhs_body(hist_ref, r_ref, h_ref, rhs_ref):
    _, dinv = _deg_dinv(hist_ref)
    rhs_ref[...] = dinv * (r_ref[...] * h_ref[...])


def _tc_rhs(hist2, r, h):
    return pl.pallas_call(
        _tc_rhs_body,
        grid=(_GRID,),
        in_specs=[
            pl.BlockSpec((_NC, _BLK, _C), lambda i: (0, i, 0)),
            pl.BlockSpec((_BLK, _C), lambda i: (i, 0)),
            pl.BlockSpec((_BLK, _C), lambda i: (i, 0)),
        ],
        out_specs=pl.BlockSpec((_BLK, _C), lambda i: (i, 0)),
        out_shape=jax.ShapeDtypeStruct((_N, _C), jnp.float32),
    )(hist2, r, h)


def _tc_conv2_body(prh_ref, hist_ref, r_ref, h_ref, u_ref, mx_ref, w2_ref,
                   b2_ref, out_ref):
    deg, dinv = _deg_dinv(hist_ref)
    two_over = 2.0 / deg
    rh = r_ref[...] * h_ref[...]
    mrh = dinv * (prh_ref[0] + prh_ref[1]) + two_over * rh
    cat = jnp.concatenate([mx_ref[...], mrh], axis=1)
    agg = jnp.dot(cat, w2_ref[...], preferred_element_type=jnp.float32)
    cnew = jnp.tanh(agg + b2_ref[...])
    u = u_ref[...]
    out_ref[...] = u * h_ref[...] + (1.0 - u) * cnew


def _tc_conv2(prhp, hist2, r, h, u, mx, W2, b2):
    return pl.pallas_call(
        _tc_conv2_body,
        grid=(_GRID,),
        in_specs=[
            pl.BlockSpec((_NC, _BLK, _C), lambda i: (0, i, 0)),
            pl.BlockSpec((_NC, _BLK, _C), lambda i: (0, i, 0)),
            pl.BlockSpec((_BLK, _C), lambda i: (i, 0)),
            pl.BlockSpec((_BLK, _C), lambda i: (i, 0)),
            pl.BlockSpec((_BLK, _C), lambda i: (i, 0)),
            pl.BlockSpec((_BLK, _C), lambda i: (i, 0)),
            pl.BlockSpec((2 * _C, _C), lambda i: (0, 0)),
            pl.BlockSpec((1, _C), lambda i: (0, 0)),
        ],
        out_specs=pl.BlockSpec((_BLK, _C), lambda i: (i, 0)),
        out_shape=jax.ShapeDtypeStruct((_N, _C), jnp.float32),
    )(prhp, hist2, r, h, u, mx, W2, b2)


def kernel(x, edge_index, edge_weight, h, W1, b1, W2, b2):
    del edge_weight
    src = edge_index[0]
    dst = edge_index[1]
    ei3 = jnp.stack([src.reshape(_E // _CH, _CH),
                     dst.reshape(_E // _CH, _CH)], axis=1)
    z128 = jnp.zeros((_RT, _C), jnp.float32)
    b1r = b1.reshape(1, 2 * _C)
    b2r = b2.reshape(1, _C)

    sc_hist = _build_sc_hist()
    sc_prop = _build_sc_prop()
    sc_prop2 = _build_sc_prop2()
    hist2 = sc_hist(ei3, z128)
    xs, hs = _tc_scale(hist2, x, h)
    table2 = jnp.concatenate([xs, hs], axis=0)
    pxh = sc_prop2(table2, src, dst, z128)
    ru, mx = _tc_conv1(pxh, hist2, x, h, W1, b1r)
    r = ru[:_N // 2].reshape(_N, _C)
    u = ru[_N // 2:].reshape(_N, _C)
    rhs = _tc_rhs(hist2, r, h)
    prhp = sc_prop(rhs, ei3, z128)
    return _tc_conv2(prhp, hist2, r, h, u, mx, W2, b2r)


# speedup vs baseline: 16.7330x; 1.0466x over previous
import functools

import jax
import jax.numpy as jnp
from jax import lax
from jax.experimental import pallas as pl
from jax.experimental.pallas import tpu as pltpu
from jax.experimental.pallas import tpu_sc as plsc

_N = 10000
_NP = 10240
_E = 320000
_C = 128
_NC = 2
_NS = 16
_NW = _NC * _NS
_EW = _E // _NW
_CH = 80
_NCHUNK = _EW // _CH
_RT = _NP // _NS
_NB = 4


def _mesh():
    return plsc.VectorSubcoreMesh(core_axis_name="c", subcore_axis_name="s")


@functools.cache
def _build_sc_hist():
    return functools.partial(
        pl.kernel,
        out_type=jax.ShapeDtypeStruct((_NC, _NP, _C), jnp.float32),
        mesh=_mesh(),
        scratch_types=[
            pltpu.VMEM((2, _CH), jnp.int32),
            pltpu.VMEM((2, _CH), jnp.int32),
            pltpu.VMEM((_CH, _C), jnp.float32),
            pltpu.VMEM_SHARED((_NP, _C), jnp.float32),
            pltpu.SemaphoreType.DMA,
            pltpu.SemaphoreType.DMA,
        ],
    )(_sc_hist_body)


def _sc_hist_body(ei_h, z128_h, out_h, eb_a, eb_b, ones_v, acc, sem_a, sem_b):
    c = lax.axis_index("c")
    s = lax.axis_index("s")
    w = c * _NS + s
    base = w * _NCHUNK

    def fill(t, _):
        i = t // (_C // 16)
        j = t % (_C // 16)
        ones_v[i, pl.ds(j * 16, 16)] = jnp.full((16,), 1.0, jnp.float32)
        return 0
    lax.fori_loop(0, _CH * (_C // 16), fill, 0)

    pltpu.sync_copy(z128_h, acc.at[pl.ds(s * _RT, _RT)])
    plsc.subcore_barrier()

    ebs = [eb_a, eb_b]
    sems = [sem_a, sem_b]
    cps = [None, None]

    def start_idx(j, b):
        cps[b] = pltpu.async_copy(ei_h.at[base + j], ebs[b], sems[b])

    start_idx(0, 0)
    for j in range(_NCHUNK):
        cur = j % 2
        if j + 1 < _NCHUNK:
            start_idx(j + 1, 1 - cur)
        cps[cur].wait()
        pltpu.sync_copy(ones_v, acc.at[ebs[cur].at[1]], add=True)

    plsc.subcore_barrier()
    pltpu.sync_copy(acc.at[pl.ds(s * _RT, _RT)],
                    out_h.at[c, pl.ds(s * _RT, _RT)])


_CHP = 160
_NFP = 62
_TAIL = _EW - _NFP * _CHP

@functools.cache
def _build_sc_prop():
    return functools.partial(
        pl.kernel,
        out_type=jax.ShapeDtypeStruct((_NC, _NP, _C), jnp.float32),
        mesh=_mesh(),
        scratch_types=(
            [pltpu.VMEM((_CHP,), jnp.int32) for _ in range(2)]
            + [pltpu.VMEM((_CHP,), jnp.int32) for _ in range(2)]
            + [pltpu.VMEM((_CHP, _C), jnp.float32) for _ in range(2)]
            + [pltpu.VMEM((_TAIL,), jnp.int32) for _ in range(2)]
            + [pltpu.VMEM_SHARED((_NP, _C), jnp.float32)]
            + [pltpu.SemaphoreType.DMA for _ in range(7)]
        ),
    )(_sc_prop_body)


def _sc_prop_body(table_h, src_h, dst_h, z128_h, out_h, *refs):
    sbs = list(refs[0:2])
    dbs = list(refs[2:4])
    rws = list(refs[4:6])
    stl, dtl = refs[6], refs[7]
    acc = refs[8]
    ssems = list(refs[9:11])
    dsems = list(refs[11:13])
    gsems = list(refs[13:15])
    tsem = refs[15]
    c = lax.axis_index("c")
    s = lax.axis_index("s")
    w = c * _NS + s
    base = w * _EW

    pltpu.sync_copy(z128_h, acc.at[pl.ds(s * _RT, _RT)])
    plsc.subcore_barrier()

    scps = [None, None]
    dcps = [None, None]
    gcps = [None, None]

    def start_idx(j, b):
        eoff = pl.multiple_of(base + j * _CHP, 8)
        scps[b] = pltpu.async_copy(src_h.at[pl.ds(eoff, _CHP)], sbs[b],
                                   ssems[b])
        dcps[b] = pltpu.async_copy(dst_h.at[pl.ds(eoff, _CHP)], dbs[b],
                                   dsems[b])

    def start_gather(b):
        scps[b].wait()
        gcps[b] = pltpu.async_copy(table_h.at[sbs[b]], rws[b], gsems[b])

    toff = pl.multiple_of(base + _NFP * _CHP, 8)
    tscp = pltpu.async_copy(src_h.at[pl.ds(toff, _TAIL)], stl, tsem)

    start_idx(0, 0)
    start_gather(0)
    start_idx(1, 1)
    for j in range(_NFP):
        cur = j % 2
        nxt = 1 - cur
        if j + 1 < _NFP:
            start_gather(nxt)
        gcps[cur].wait()
        dcps[cur].wait()
        pltpu.sync_copy(rws[cur], acc.at[dbs[cur]], add=True)
        if j + 2 < _NFP:
            start_idx(j + 2, cur)

    tscp.wait()
    tgcp = pltpu.async_copy(table_h.at[stl], rws[0].at[pl.ds(0, _TAIL)],
                            tsem)
    pltpu.sync_copy(dst_h.at[pl.ds(toff, _TAIL)], dtl)
    tgcp.wait()
    pltpu.sync_copy(rws[0].at[pl.ds(0, _TAIL)], acc.at[dtl], add=True)

    plsc.subcore_barrier()
    pltpu.sync_copy(acc.at[pl.ds(s * _RT, _RT)],
                    out_h.at[c, pl.ds(s * _RT, _RT)])


_CH2 = 160
_NCH2 = _E // _CH2 // _NS


@functools.cache
def _build_sc_prop2():
    return functools.partial(
        pl.kernel,
        out_type=jax.ShapeDtypeStruct((_NC, _NP, _C), jnp.float32),
        mesh=_mesh(),
        scratch_types=(
            [pltpu.VMEM((_CH2,), jnp.int32) for _ in range(2)]
            + [pltpu.VMEM((_CH2,), jnp.int32) for _ in range(2)]
            + [pltpu.VMEM((_CH2,), jnp.int32) for _ in range(2)]
            + [pltpu.VMEM((_CH2, _C), jnp.float32) for _ in range(2)]
            + [pltpu.VMEM_SHARED((_NP, _C), jnp.float32)]
            + [pltpu.SemaphoreType.DMA for _ in range(6)]
        ),
    )(_sc_prop2_body)


def _sc_prop2_body(table2_h, src_h, dst_h, z128_h, out_h, *refs):
    sbs = list(refs[0:2])
    s2s = list(refs[2:4])
    dbs = list(refs[4:6])
    rws = list(refs[6:8])
    acc = refs[8]
    ssems = list(refs[9:11])
    dsems = list(refs[11:13])
    gsems = list(refs[13:15])
    c = lax.axis_index("c")
    s = lax.axis_index("s")
    base = s * _NCH2
    off = c * _N

    pltpu.sync_copy(z128_h, acc.at[pl.ds(s * _RT, _RT)])
    plsc.subcore_barrier()

    scps = [None, None]
    dcps = [None, None]
    gcps = [None, None]

    def start_idx(j, b):
        eoff = pl.multiple_of((base + j) * _CH2, 8)
        scps[b] = pltpu.async_copy(src_h.at[pl.ds(eoff, _CH2)], sbs[b],
                                   ssems[b])
        dcps[b] = pltpu.async_copy(dst_h.at[pl.ds(eoff, _CH2)], dbs[b],
                                   dsems[b])

    def adjust(b):
        scps[b].wait()
        for k in range(_CH2 // 16):
            s2s[b][pl.ds(k * 16, 16)] = sbs[b][pl.ds(k * 16, 16)] + off

    def start_gather(b):
        gcps[b] = pltpu.async_copy(table2_h.at[s2s[b]], rws[b], gsems[b])

    start_idx(0, 0)
    adjust(0)
    start_gather(0)
    start_idx(1, 1)
    for j in range(_NCH2):
        cur = j % 2
        nxt = 1 - cur
        if j + 1 < _NCH2:
            adjust(nxt)
            start_gather(nxt)
        gcps[cur].wait()
        dcps[cur].wait()
        pltpu.sync_copy(rws[cur], acc.at[dbs[cur]], add=True)
        if j + 2 < _NCH2:
            start_idx(j + 2, cur)

    plsc.subcore_barrier()
    pltpu.sync_copy(acc.at[pl.ds(s * _RT, _RT)],
                    out_h.at[c, pl.ds(s * _RT, _RT)])


_BLK = 1000
_GRID = _N // _BLK


def _deg_dinv(hist_ref):
    cnt = (hist_ref[0][:, 0:1] + hist_ref[1][:, 0:1]).astype(jnp.float32)
    deg = cnt + 2.0
    return deg, lax.rsqrt(deg)


def _tc_scale_body(hist_ref, x_ref, h_ref, xs_ref, hs_ref):
    _, dinv = _deg_dinv(hist_ref)
    xs_ref[...] = x_ref[...] * dinv
    hs_ref[...] = h_ref[...] * dinv


def _tc_scale(hist2, x, h):
    return pl.pallas_call(
        _tc_scale_body,
        grid=(_GRID,),
        in_specs=[
            pl.BlockSpec((_NC, _BLK, _C), lambda i: (0, i, 0)),
            pl.BlockSpec((_BLK, _C), lambda i: (i, 0)),
            pl.BlockSpec((_BLK, _C), lambda i: (i, 0)),
        ],
        out_specs=[
            pl.BlockSpec((_BLK, _C), lambda i: (i, 0)),
            pl.BlockSpec((_BLK, _C), lambda i: (i, 0)),
        ],
        out_shape=[
            jax.ShapeDtypeStruct((_N, _C), jnp.float32),
            jax.ShapeDtypeStruct((_N, _C), jnp.float32),
        ],
    )(hist2, x, h)


def _tc_conv1_body(pxh_ref, hist_ref, x_ref, h_ref, w1_ref, b1_ref,
                   ru_ref, mx_ref):
    deg, dinv = _deg_dinv(hist_ref)
    two_over = 2.0 / deg
    mx = dinv * pxh_ref[0] + two_over * x_ref[...]
    mh = dinv * pxh_ref[1] + two_over * h_ref[...]
    cat = jnp.concatenate([mx, mh], axis=1)
    agg = jnp.dot(cat, w1_ref[...], preferred_element_type=jnp.float32)
    ru_ref[...] = jax.nn.sigmoid(agg + b1_ref[...])
    mx_ref[...] = mx


def _tc_conv1(pxh, hist2, x, h, W1, b1):
    return pl.pallas_call(
        _tc_conv1_body,
        grid=(_GRID,),
        in_specs=[
            pl.BlockSpec((_NC, _BLK, _C), lambda i: (0, i, 0)),
            pl.BlockSpec((_NC, _BLK, _C), lambda i: (0, i, 0)),
            pl.BlockSpec((_BLK, _C), lambda i: (i, 0)),
            pl.BlockSpec((_BLK, _C), lambda i: (i, 0)),
            pl.BlockSpec((2 * _C, 2 * _C), lambda i: (0, 0)),
            pl.BlockSpec((1, 2 * _C), lambda i: (0, 0)),
        ],
        out_specs=[
            pl.BlockSpec((_BLK, 2 * _C), lambda i: (i, 0)),
            pl.BlockSpec((_BLK, _C), lambda i: (i, 0)),
        ],
        out_shape=[
            jax.ShapeDtypeStruct((_N, 2 * _C), jnp.float32),
            jax.ShapeDtypeStruct((_N, _C), jnp.float32),
        ],
    )(pxh, hist2, x, h, W1, b1)


def _tc_rhs_body(hist_ref, r_ref, h_ref, rhs_ref):
    _, dinv = _deg_dinv(hist_ref)
    rhs_ref[...] = dinv * (r_ref[...] * h_ref[...])


def _tc_rhs(hist2, r, h):
    return pl.pallas_call(
        _tc_rhs_body,
        grid=(_GRID,),
        in_specs=[
            pl.BlockSpec((_NC, _BLK, _C), lambda i: (0, i, 0)),
            pl.BlockSpec((_BLK, _C), lambda i: (i, 0)),
            pl.BlockSpec((_BLK, _C), lambda i: (i, 0)),
        ],
        out_specs=pl.BlockSpec((_BLK, _C), lambda i: (i, 0)),
        out_shape=jax.ShapeDtypeStruct((_N, _C), jnp.float32),
    )(hist2, r, h)


def _tc_conv2_body(prh_ref, hist_ref, r_ref, h_ref, u_ref, mx_ref, w2_ref,
                   b2_ref, out_ref):
    deg, dinv = _deg_dinv(hist_ref)
    two_over = 2.0 / deg
    rh = r_ref[...] * h_ref[...]
    mrh = dinv * (prh_ref[0] + prh_ref[1]) + two_over * rh
    cat = jnp.concatenate([mx_ref[...], mrh], axis=1)
    agg = jnp.dot(cat, w2_ref[...], preferred_element_type=jnp.float32)
    cnew = jnp.tanh(agg + b2_ref[...])
    u = u_ref[...]
    out_ref[...] = u * h_ref[...] + (1.0 - u) * cnew


def _tc_conv2(prhp, hist2, r, h, u, mx, W2, b2):
    return pl.pallas_call(
        _tc_conv2_body,
        grid=(_GRID,),
        in_specs=[
            pl.BlockSpec((_NC, _BLK, _C), lambda i: (0, i, 0)),
            pl.BlockSpec((_NC, _BLK, _C), lambda i: (0, i, 0)),
            pl.BlockSpec((_BLK, _C), lambda i: (i, 0)),
            pl.BlockSpec((_BLK, _C), lambda i: (i, 0)),
            pl.BlockSpec((_BLK, _C), lambda i: (i, 0)),
            pl.BlockSpec((_BLK, _C), lambda i: (i, 0)),
            pl.BlockSpec((2 * _C, _C), lambda i: (0, 0)),
            pl.BlockSpec((1, _C), lambda i: (0, 0)),
        ],
        out_specs=pl.BlockSpec((_BLK, _C), lambda i: (i, 0)),
        out_shape=jax.ShapeDtypeStruct((_N, _C), jnp.float32),
    )(prhp, hist2, r, h, u, mx, W2, b2)


def kernel(x, edge_index, edge_weight, h, W1, b1, W2, b2):
    del edge_weight
    src = edge_index[0]
    dst = edge_index[1]
    ei3 = jnp.stack([src.reshape(_E // _CH, _CH),
                     dst.reshape(_E // _CH, _CH)], axis=1)
    z128 = jnp.zeros((_RT, _C), jnp.float32)
    b1r = b1.reshape(1, 2 * _C)
    b2r = b2.reshape(1, _C)

    sc_hist = _build_sc_hist()
    sc_prop = _build_sc_prop()
    sc_prop2 = _build_sc_prop2()
    hist2 = sc_hist(ei3, z128)
    xs, hs = _tc_scale(hist2, x, h)
    table2 = jnp.concatenate([xs, hs], axis=0)
    pxh = sc_prop2(table2, src, dst, z128)
    ru, mx = _tc_conv1(pxh, hist2, x, h, W1, b1r)
    r = ru[:_N // 2].reshape(_N, _C)
    u = ru[_N // 2:].reshape(_N, _C)
    rhs = _tc_rhs(hist2, r, h)
    prhp = sc_prop(rhs, src, dst, z128)
    return _tc_conv2(prhp, hist2, r, h, u, mx, W2, b2r)


# speedup vs baseline: 16.9345x; 1.0120x over previous
import functools

import jax
import jax.numpy as jnp
from jax import lax
from jax.experimental import pallas as pl
from jax.experimental.pallas import tpu as pltpu
from jax.experimental.pallas import tpu_sc as plsc

_N = 10000
_NP = 10240
_E = 320000
_C = 128
_NC = 2
_NS = 16
_NW = _NC * _NS
_EW = _E // _NW
_CH = 80
_NCHUNK = _EW // _CH
_RT = _NP // _NS
_NB = 4


def _mesh():
    return plsc.VectorSubcoreMesh(core_axis_name="c", subcore_axis_name="s")


@functools.cache
def _build_sc_hist():
    return functools.partial(
        pl.kernel,
        out_type=jax.ShapeDtypeStruct((_NC, _NP, _C), jnp.float32),
        mesh=_mesh(),
        scratch_types=(
            [pltpu.VMEM((160,), jnp.int32) for _ in range(2)]
            + [pltpu.VMEM((80,), jnp.int32)]
            + [pltpu.VMEM((160, _C), jnp.float32)]
            + [pltpu.VMEM_SHARED((_NP, _C), jnp.float32)]
            + [pltpu.SemaphoreType.DMA for _ in range(3)]
        ),
    )(_sc_hist_body)


def _sc_hist_body(dst_h, z128_h, out_h, *refs):
    dbs = list(refs[0:2])
    dtl = refs[2]
    ones_v = refs[3]
    acc = refs[4]
    dsems = list(refs[5:7])
    tsem = refs[7]
    c = lax.axis_index("c")
    s = lax.axis_index("s")
    w = c * _NS + s
    base = w * _EW

    def fill(t, _):
        i = t // (_C // 16)
        j = t % (_C // 16)
        ones_v[i, pl.ds(j * 16, 16)] = jnp.full((16,), 1.0, jnp.float32)
        return 0
    lax.fori_loop(0, 160 * (_C // 16), fill, 0)

    pltpu.sync_copy(z128_h, acc.at[pl.ds(s * _RT, _RT)])
    plsc.subcore_barrier()

    dcps = [None, None]

    def start_idx(j, b):
        eoff = pl.multiple_of(base + j * 160, 8)
        dcps[b] = pltpu.async_copy(dst_h.at[pl.ds(eoff, 160)], dbs[b],
                                   dsems[b])

    toff = pl.multiple_of(base + 62 * 160, 8)
    tcp = pltpu.async_copy(dst_h.at[pl.ds(toff, 80)], dtl, tsem)

    start_idx(0, 0)
    start_idx(1, 1)
    for j in range(62):
        cur = j % 2
        dcps[cur].wait()
        pltpu.sync_copy(ones_v, acc.at[dbs[cur]], add=True)
        if j + 2 < 62:
            start_idx(j + 2, cur)

    tcp.wait()
    pltpu.sync_copy(ones_v.at[pl.ds(0, 80)], acc.at[dtl], add=True)

    plsc.subcore_barrier()
    pltpu.sync_copy(acc.at[pl.ds(s * _RT, _RT)],
                    out_h.at[c, pl.ds(s * _RT, _RT)])


_CHP = 160
_NFP = 62
_TAIL = _EW - _NFP * _CHP

@functools.cache
def _build_sc_prop():
    return functools.partial(
        pl.kernel,
        out_type=jax.ShapeDtypeStruct((_NC, _NP, _C), jnp.float32),
        mesh=_mesh(),
        scratch_types=(
            [pltpu.VMEM((_CHP,), jnp.int32) for _ in range(2)]
            + [pltpu.VMEM((_CHP,), jnp.int32) for _ in range(2)]
            + [pltpu.VMEM((_CHP, _C), jnp.float32) for _ in range(2)]
            + [pltpu.VMEM((_TAIL,), jnp.int32) for _ in range(2)]
            + [pltpu.VMEM_SHARED((_NP, _C), jnp.float32)]
            + [pltpu.SemaphoreType.DMA for _ in range(7)]
        ),
    )(_sc_prop_body)


def _sc_prop_body(table_h, src_h, dst_h, z128_h, out_h, *refs):
    sbs = list(refs[0:2])
    dbs = list(refs[2:4])
    rws = list(refs[4:6])
    stl, dtl = refs[6], refs[7]
    acc = refs[8]
    ssems = list(refs[9:11])
    dsems = list(refs[11:13])
    gsems = list(refs[13:15])
    tsem = refs[15]
    c = lax.axis_index("c")
    s = lax.axis_index("s")
    w = c * _NS + s
    base = w * _EW

    pltpu.sync_copy(z128_h, acc.at[pl.ds(s * _RT, _RT)])
    plsc.subcore_barrier()

    scps = [None, None]
    dcps = [None, None]
    gcps = [None, None]

    def start_idx(j, b):
        eoff = pl.multiple_of(base + j * _CHP, 8)
        scps[b] = pltpu.async_copy(src_h.at[pl.ds(eoff, _CHP)], sbs[b],
                                   ssems[b])
        dcps[b] = pltpu.async_copy(dst_h.at[pl.ds(eoff, _CHP)], dbs[b],
                                   dsems[b])

    def start_gather(b):
        scps[b].wait()
        gcps[b] = pltpu.async_copy(table_h.at[sbs[b]], rws[b], gsems[b])

    toff = pl.multiple_of(base + _NFP * _CHP, 8)
    tscp = pltpu.async_copy(src_h.at[pl.ds(toff, _TAIL)], stl, tsem)

    start_idx(0, 0)
    start_gather(0)
    start_idx(1, 1)
    for j in range(_NFP):
        cur = j % 2
        nxt = 1 - cur
        if j + 1 < _NFP:
            start_gather(nxt)
        gcps[cur].wait()
        dcps[cur].wait()
        pltpu.sync_copy(rws[cur], acc.at[dbs[cur]], add=True)
        if j + 2 < _NFP:
            start_idx(j + 2, cur)

    tscp.wait()
    tgcp = pltpu.async_copy(table_h.at[stl], rws[0].at[pl.ds(0, _TAIL)],
                            tsem)
    pltpu.sync_copy(dst_h.at[pl.ds(toff, _TAIL)], dtl)
    tgcp.wait()
    pltpu.sync_copy(rws[0].at[pl.ds(0, _TAIL)], acc.at[dtl], add=True)

    plsc.subcore_barrier()
    pltpu.sync_copy(acc.at[pl.ds(s * _RT, _RT)],
                    out_h.at[c, pl.ds(s * _RT, _RT)])


_CH2 = 160
_NCH2 = _E // _CH2 // _NS


@functools.cache
def _build_sc_prop2():
    return functools.partial(
        pl.kernel,
        out_type=jax.ShapeDtypeStruct((_NC, _NP, _C), jnp.float32),
        mesh=_mesh(),
        scratch_types=(
            [pltpu.VMEM((_CH2,), jnp.int32) for _ in range(2)]
            + [pltpu.VMEM((_CH2,), jnp.int32) for _ in range(2)]
            + [pltpu.VMEM((_CH2,), jnp.int32) for _ in range(2)]
            + [pltpu.VMEM((_CH2, _C), jnp.float32) for _ in range(2)]
            + [pltpu.VMEM_SHARED((_NP, _C), jnp.float32)]
            + [pltpu.SemaphoreType.DMA for _ in range(6)]
        ),
    )(_sc_prop2_body)


def _sc_prop2_body(table2_h, src_h, dst_h, z128_h, out_h, *refs):
    sbs = list(refs[0:2])
    s2s = list(refs[2:4])
    dbs = list(refs[4:6])
    rws = list(refs[6:8])
    acc = refs[8]
    ssems = list(refs[9:11])
    dsems = list(refs[11:13])
    gsems = list(refs[13:15])
    c = lax.axis_index("c")
    s = lax.axis_index("s")
    base = s * _NCH2
    off = c * _N

    pltpu.sync_copy(z128_h, acc.at[pl.ds(s * _RT, _RT)])
    plsc.subcore_barrier()

    scps = [None, None]
    dcps = [None, None]
    gcps = [None, None]

    def start_idx(j, b):
        eoff = pl.multiple_of((base + j) * _CH2, 8)
        scps[b] = pltpu.async_copy(src_h.at[pl.ds(eoff, _CH2)], sbs[b],
                                   ssems[b])
        dcps[b] = pltpu.async_copy(dst_h.at[pl.ds(eoff, _CH2)], dbs[b],
                                   dsems[b])

    def adjust(b):
        scps[b].wait()
        for k in range(_CH2 // 16):
            s2s[b][pl.ds(k * 16, 16)] = sbs[b][pl.ds(k * 16, 16)] + off

    def start_gather(b):
        gcps[b] = pltpu.async_copy(table2_h.at[s2s[b]], rws[b], gsems[b])

    start_idx(0, 0)
    adjust(0)
    start_gather(0)
    start_idx(1, 1)
    for j in range(_NCH2):
        cur = j % 2
        nxt = 1 - cur
        if j + 1 < _NCH2:
            adjust(nxt)
            start_gather(nxt)
        gcps[cur].wait()
        dcps[cur].wait()
        pltpu.sync_copy(rws[cur], acc.at[dbs[cur]], add=True)
        if j + 2 < _NCH2:
            start_idx(j + 2, cur)

    plsc.subcore_barrier()
    pltpu.sync_copy(acc.at[pl.ds(s * _RT, _RT)],
                    out_h.at[c, pl.ds(s * _RT, _RT)])


_BLK = 1000
_GRID = _N // _BLK


def _deg_dinv(hist_ref):
    cnt = (hist_ref[0][:, 0:1] + hist_ref[1][:, 0:1]).astype(jnp.float32)
    deg = cnt + 2.0
    return deg, lax.rsqrt(deg)


def _tc_scale_body(hist_ref, x_ref, h_ref, xs_ref, hs_ref):
    _, dinv = _deg_dinv(hist_ref)
    xs_ref[...] = x_ref[...] * dinv
    hs_ref[...] = h_ref[...] * dinv


def _tc_scale(hist2, x, h):
    return pl.pallas_call(
        _tc_scale_body,
        grid=(_GRID,),
        in_specs=[
            pl.BlockSpec((_NC, _BLK, _C), lambda i: (0, i, 0)),
            pl.BlockSpec((_BLK, _C), lambda i: (i, 0)),
            pl.BlockSpec((_BLK, _C), lambda i: (i, 0)),
        ],
        out_specs=[
            pl.BlockSpec((_BLK, _C), lambda i: (i, 0)),
            pl.BlockSpec((_BLK, _C), lambda i: (i, 0)),
        ],
        out_shape=[
            jax.ShapeDtypeStruct((_N, _C), jnp.float32),
            jax.ShapeDtypeStruct((_N, _C), jnp.float32),
        ],
    )(hist2, x, h)


def _tc_conv1_body(pxh_ref, hist_ref, x_ref, h_ref, w1_ref, b1_ref,
                   ru_ref, mx_ref):
    deg, dinv = _deg_dinv(hist_ref)
    two_over = 2.0 / deg
    mx = dinv * pxh_ref[0] + two_over * x_ref[...]
    mh = dinv * pxh_ref[1] + two_over * h_ref[...]
    cat = jnp.concatenate([mx, mh], axis=1)
    agg = jnp.dot(cat, w1_ref[...], preferred_element_type=jnp.float32)
    ru_ref[...] = jax.nn.sigmoid(agg + b1_ref[...])
    mx_ref[...] = mx


def _tc_conv1(pxh, hist2, x, h, W1, b1):
    return pl.pallas_call(
        _tc_conv1_body,
        grid=(_GRID,),
        in_specs=[
            pl.BlockSpec((_NC, _BLK, _C), lambda i: (0, i, 0)),
            pl.BlockSpec((_NC, _BLK, _C), lambda i: (0, i, 0)),
            pl.BlockSpec((_BLK, _C), lambda i: (i, 0)),
            pl.BlockSpec((_BLK, _C), lambda i: (i, 0)),
            pl.BlockSpec((2 * _C, 2 * _C), lambda i: (0, 0)),
            pl.BlockSpec((1, 2 * _C), lambda i: (0, 0)),
        ],
        out_specs=[
            pl.BlockSpec((_BLK, 2 * _C), lambda i: (i, 0)),
            pl.BlockSpec((_BLK, _C), lambda i: (i, 0)),
        ],
        out_shape=[
            jax.ShapeDtypeStruct((_N, 2 * _C), jnp.float32),
            jax.ShapeDtypeStruct((_N, _C), jnp.float32),
        ],
    )(pxh, hist2, x, h, W1, b1)


def _tc_rhs_body(hist_ref, r_ref, h_ref, rhs_ref):
    _, dinv = _deg_dinv(hist_ref)
    rhs_ref[...] = dinv * (r_ref[...] * h_ref[...])


def _tc_rhs(hist2, r, h):
    return pl.pallas_call(
        _tc_rhs_body,
        grid=(_GRID,),
        in_specs=[
            pl.BlockSpec((_NC, _BLK, _C), lambda i: (0, i, 0)),
            pl.BlockSpec((_BLK, _C), lambda i: (i, 0)),
            pl.BlockSpec((_BLK, _C), lambda i: (i, 0)),
        ],
        out_specs=pl.BlockSpec((_BLK, _C), lambda i: (i, 0)),
        out_shape=jax.ShapeDtypeStruct((_N, _C), jnp.float32),
    )(hist2, r, h)


def _tc_conv2_body(prh_ref, hist_ref, r_ref, h_ref, u_ref, mx_ref, w2_ref,
                   b2_ref, out_ref):
    deg, dinv = _deg_dinv(hist_ref)
    two_over = 2.0 / deg
    rh = r_ref[...] * h_ref[...]
    mrh = dinv * (prh_ref[0] + prh_ref[1]) + two_over * rh
    cat = jnp.concatenate([mx_ref[...], mrh], axis=1)
    agg = jnp.dot(cat, w2_ref[...], preferred_element_type=jnp.float32)
    cnew = jnp.tanh(agg + b2_ref[...])
    u = u_ref[...]
    out_ref[...] = u * h_ref[...] + (1.0 - u) * cnew


def _tc_conv2(prhp, hist2, r, h, u, mx, W2, b2):
    return pl.pallas_call(
        _tc_conv2_body,
        grid=(_GRID,),
        in_specs=[
            pl.BlockSpec((_NC, _BLK, _C), lambda i: (0, i, 0)),
            pl.BlockSpec((_NC, _BLK, _C), lambda i: (0, i, 0)),
            pl.BlockSpec((_BLK, _C), lambda i: (i, 0)),
            pl.BlockSpec((_BLK, _C), lambda i: (i, 0)),
            pl.BlockSpec((_BLK, _C), lambda i: (i, 0)),
            pl.BlockSpec((_BLK, _C), lambda i: (i, 0)),
            pl.BlockSpec((2 * _C, _C), lambda i: (0, 0)),
            pl.BlockSpec((1, _C), lambda i: (0, 0)),
        ],
        out_specs=pl.BlockSpec((_BLK, _C), lambda i: (i, 0)),
        out_shape=jax.ShapeDtypeStruct((_N, _C), jnp.float32),
    )(prhp, hist2, r, h, u, mx, W2, b2)


def kernel(x, edge_index, edge_weight, h, W1, b1, W2, b2):
    del edge_weight
    src = edge_index[0]
    dst = edge_index[1]
    z128 = jnp.zeros((_RT, _C), jnp.float32)
    b1r = b1.reshape(1, 2 * _C)
    b2r = b2.reshape(1, _C)

    sc_hist = _build_sc_hist()
    sc_prop = _build_sc_prop()
    sc_prop2 = _build_sc_prop2()
    hist2 = sc_hist(dst, z128)
    xs, hs = _tc_scale(hist2, x, h)
    table2 = jnp.concatenate([xs, hs], axis=0)
    pxh = sc_prop2(table2, src, dst, z128)
    ru, mx = _tc_conv1(pxh, hist2, x, h, W1, b1r)
    r = ru[:_N // 2].reshape(_N, _C)
    u = ru[_N // 2:].reshape(_N, _C)
    rhs = _tc_rhs(hist2, r, h)
    prhp = sc_prop(rhs, src, dst, z128)
    return _tc_conv2(prhp, hist2, r, h, u, mx, W2, b2r)
